# baseline (speedup 1.0000x reference)
import functools

import jax
import jax.numpy as jnp
from jax import lax
from jax.experimental import pallas as pl
from jax.experimental.pallas import tpu as pltpu
from jax.experimental.pallas import tpu_sc as plsc

NC = 2
NS = 16
NW = NC * NS

K = 8
H = 4


@functools.lru_cache(maxsize=None)
def _build(B, V, D):
    bpw = B // NW
    chunks = bpw // K
    assert B % (NW * K) == 0 and chunks >= 4 and chunks % 2 == 0

    mesh = plsc.VectorSubcoreMesh(core_axis_name="c", subcore_axis_name="s")

    @functools.partial(
        pl.kernel,
        mesh=mesh,
        out_type=jax.ShapeDtypeStruct((B, D), jnp.float32),
        scratch_types=(
            [pltpu.VMEM((bpw,), jnp.int32),
             pltpu.VMEM((2, K, D), jnp.float32),
             pltpu.MemorySpace.VMEM_SHARED((NS, 2, H, D), jnp.float32)]
            + [pltpu.SemaphoreType.DMA] * 6
        ),
    )
    def emb(idx_hbm, tab_hbm, out_hbm, idx_v, bufs, shr, *sems):
        gsems = sems[0:2]
        xsems = sems[2:4]
        wsems = sems[4:6]
        wid = lax.axis_index("s") * NC + lax.axis_index("c")
        sid = lax.axis_index("s")
        base = wid * bpw
        pltpu.sync_copy(idx_hbm.at[pl.ds(base, bpw)], idx_v)

        def start_gather(b, g):
            off = pl.multiple_of(g * K, 8)
            pltpu.async_copy(
                tab_hbm.at[idx_v.at[pl.ds(off, K)]], bufs.at[b], gsems[b])

        def wait_gather(b):
            pltpu.make_async_copy(
                tab_hbm.at[idx_v.at[pl.ds(0, K)]], bufs.at[b], gsems[b]).wait()

        def start_x(b, h):
            src = bufs.at[b].at[pl.ds(h * H, H)]
            pltpu.async_copy(src, shr.at[sid, h], xsems[h])

        def wait_x(b, h):
            src = bufs.at[b].at[pl.ds(h * H, H)]
            pltpu.make_async_copy(src, shr.at[sid, h], xsems[h]).wait()

        def start_write(h, g):
            off = pl.multiple_of(base + g * K + h * H, H)
            pltpu.async_copy(shr.at[sid, h], out_hbm.at[pl.ds(off, H)],
                             wsems[h])

        def wait_write(h):
            pltpu.make_async_copy(shr.at[sid, h],
                                  out_hbm.at[pl.ds(0, H)], wsems[h]).wait()

        def process(b, g, first=False, nxt=True):
            wait_gather(b)
            for h in range(K // H):
                if not first:
                    wait_write(h)
                start_x(b, h)
            for h in range(K // H):
                wait_x(b, h)
                start_write(h, g)
            if nxt:
                start_gather(b, g + 2)

        start_gather(0, 0)
        start_gather(1, 1)
        process(0, 0, first=True)

        @pl.loop(0, (chunks - 4) // 2)
        def _(i):
            process(1, 2 * i + 1)
            process(0, 2 * i + 2)

        process(1, chunks - 3)
        process(0, chunks - 2, nxt=False)
        process(1, chunks - 1, nxt=False)
        for h in range(K // H):
            wait_write(h)

    return emb


def kernel(input_ids, embed_weight):
    V, D = embed_weight.shape
    idx = input_ids.reshape(-1).astype(jnp.int32)
    B = idx.shape[0]
    out = _build(B, V, D)(idx, embed_weight)
    return out.reshape(input_ids.shape + (D,))


# speedup vs baseline: 1.0043x; 1.0043x over previous
import functools

import jax
import jax.numpy as jnp
from jax import lax
from jax.experimental import pallas as pl
from jax.experimental.pallas import tpu as pltpu
from jax.experimental.pallas import tpu_sc as plsc

NC = 2
NS = 16
NW = NC * NS

K = 8
H = 4


@functools.lru_cache(maxsize=None)
def _build(B, V, D):
    bpw = B // NW
    chunks = bpw // K
    assert B % (NW * K) == 0 and chunks >= 4 and chunks % 2 == 0

    mesh = plsc.VectorSubcoreMesh(core_axis_name="c", subcore_axis_name="s")

    @functools.partial(
        pl.kernel,
        mesh=mesh,
        out_type=jax.ShapeDtypeStruct((B, D), jnp.float32),
        scratch_types=(
            [pltpu.VMEM((bpw,), jnp.int32),
             pltpu.VMEM((2, K, D), jnp.float32),
             pltpu.MemorySpace.VMEM_SHARED((NS, 2, H, D), jnp.float32)]
            + [pltpu.SemaphoreType.DMA] * 5
        ),
    )
    def emb(idx_hbm, tab_hbm, out_hbm, idx_v, bufs, shr, *sems):
        gsems = sems[0:2]
        xsem = sems[2]
        wsems = sems[3:5]
        wid = lax.axis_index("s") * NC + lax.axis_index("c")
        sid = lax.axis_index("s")
        base = wid * bpw
        pltpu.sync_copy(idx_hbm.at[pl.ds(base, bpw)], idx_v)

        def start_gather(b, g):
            off = pl.multiple_of(g * K, 8)
            pltpu.async_copy(
                tab_hbm.at[idx_v.at[pl.ds(off, K)]], bufs.at[b], gsems[b])

        def wait_gather(b):
            pltpu.make_async_copy(
                tab_hbm.at[idx_v.at[pl.ds(0, K)]], bufs.at[b], gsems[b]).wait()

        def do_x(b, h):
            src = bufs.at[b].at[pl.ds(h * H, H)]
            pltpu.async_copy(src, shr.at[sid, h], xsem)
            pltpu.make_async_copy(src, shr.at[sid, h], xsem).wait()

        def start_write(h, g):
            off = pl.multiple_of(base + g * K + h * H, H)
            pltpu.async_copy(shr.at[sid, h], out_hbm.at[pl.ds(off, H)],
                             wsems[h])

        def wait_write(h):
            pltpu.make_async_copy(shr.at[sid, h],
                                  out_hbm.at[pl.ds(0, H)], wsems[h]).wait()

        def process(b, g, first=False, nxt=True):
            wait_gather(b)
            for h in range(K // H):
                if not first:
                    wait_write(h)
                do_x(b, h)
                start_write(h, g)
            if nxt:
                start_gather(b, g + 2)

        start_gather(0, 0)
        start_gather(1, 1)
        process(0, 0, first=True)

        @pl.loop(0, (chunks - 4) // 2)
        def _(i):
            process(1, 2 * i + 1)
            process(0, 2 * i + 2)

        process(1, chunks - 3)
        process(0, chunks - 2, nxt=False)
        process(1, chunks - 1, nxt=False)
        for h in range(K // H):
            wait_write(h)

    return emb


def kernel(input_ids, embed_weight):
    V, D = embed_weight.shape
    idx = input_ids.reshape(-1).astype(jnp.int32)
    B = idx.shape[0]
    out = _build(B, V, D)(idx, embed_weight)
    return out.reshape(input_ids.shape + (D,))
